# baseline (device time: 2768463 ns/iter reference)
import jax
import jax.numpy as jnp
from jax import lax
from jax.experimental import pallas as pl
from jax.experimental.pallas import tpu as pltpu

N_DEV = 8

RING = [0, 1, 2, 3, 7, 6, 5, 4]
POS_BY_ID = [RING.index(i) for i in range(N_DEV)]
RIGHT_BY_ID = [RING[(RING.index(i) + 1) % N_DEV] for i in range(N_DEV)]
LEFT_BY_ID = [RING[(RING.index(i) - 1) % N_DEV] for i in range(N_DEV)]


def _lut(idx, table):
    out = jnp.int32(table[0])
    for k in range(1, len(table)):
        out = jnp.where(idx == k, jnp.int32(table[k]), out)
    return out


def _gelu(y):
    c = 0.7978845608028654
    return 0.5 * y * (1.0 + jnp.tanh(c * (y + 0.044715 * y * y * y)))


def kernel(x, w_mat):
    partial = jnp.dot(x, w_mat, preferred_element_type=jnp.float32)
    m, n = partial.shape
    ch = m // N_DEV

    def body(p_ref, out_ref, send_buf, recv_buf, local_buf,
             send_sems, recv_sems, load_sem, store_sem, ack_sem):
        me = lax.axis_index("i")
        my_pos = _lut(me, POS_BY_ID)
        right = _lut(me, RIGHT_BY_ID)
        left = _lut(me, LEFT_BY_ID)

        barrier_sem = pltpu.get_barrier_semaphore()
        for nbr in (left, right):
            pl.semaphore_signal(barrier_sem, inc=1, device_id=(nbr,),
                                device_id_type=pl.DeviceIdType.MESH)
        pl.semaphore_wait(barrier_sem, 2)

        def rows(c):
            return pl.ds(c * ch, ch)

        cp = pltpu.make_async_copy(p_ref.at[rows(my_pos)], send_buf, load_sem)
        cp.start()
        cp.wait()

        for s in range(N_DEV - 1):
            rdma = pltpu.make_async_remote_copy(
                src_ref=send_buf, dst_ref=recv_buf,
                send_sem=send_sems.at[s], recv_sem=recv_sems.at[s],
                device_id=(right,), device_id_type=pl.DeviceIdType.MESH)
            if s > 0:
                pl.semaphore_wait(ack_sem, 1)
            rdma.start()
            nxt = jnp.mod(my_pos - s - 1, N_DEV)
            cp = pltpu.make_async_copy(p_ref.at[rows(nxt)], local_buf, load_sem)
            cp.start()
            rdma.wait()
            cp.wait()
            send_buf[...] = recv_buf[...] + local_buf[...]
            pl.semaphore_signal(ack_sem, inc=1, device_id=(left,),
                                device_id_type=pl.DeviceIdType.MESH)

        send_buf[...] = _gelu(send_buf[...])
        q = jnp.mod(my_pos + 1, N_DEV)
        cp = pltpu.make_async_copy(send_buf, out_ref.at[rows(q)], store_sem)
        cp.start()
        cp.wait()

        for t in range(N_DEV - 1):
            rdma = pltpu.make_async_remote_copy(
                src_ref=send_buf, dst_ref=recv_buf,
                send_sem=send_sems.at[N_DEV - 1 + t],
                recv_sem=recv_sems.at[N_DEV - 1 + t],
                device_id=(right,), device_id_type=pl.DeviceIdType.MESH)
            pl.semaphore_wait(ack_sem, 1)
            rdma.start()
            rdma.wait()
            send_buf[...] = recv_buf[...]
            if t < N_DEV - 2:
                pl.semaphore_signal(ack_sem, inc=1, device_id=(left,),
                                    device_id_type=pl.DeviceIdType.MESH)
            dst = jnp.mod(my_pos - t, N_DEV)
            cp = pltpu.make_async_copy(send_buf, out_ref.at[rows(dst)], store_sem)
            cp.start()
            cp.wait()

    n_sems = 2 * (N_DEV - 1)
    return pl.pallas_call(
        body,
        out_shape=jax.ShapeDtypeStruct((m, n), jnp.float32),
        in_specs=[pl.BlockSpec(memory_space=pltpu.MemorySpace.HBM)],
        out_specs=pl.BlockSpec(memory_space=pltpu.MemorySpace.HBM),
        scratch_shapes=[
            pltpu.VMEM((ch, n), jnp.float32),
            pltpu.VMEM((ch, n), jnp.float32),
            pltpu.VMEM((ch, n), jnp.float32),
            pltpu.SemaphoreType.DMA((n_sems,)),
            pltpu.SemaphoreType.DMA((n_sems,)),
            pltpu.SemaphoreType.DMA,
            pltpu.SemaphoreType.DMA,
            pltpu.SemaphoreType.REGULAR,
        ],
        compiler_params=pltpu.CompilerParams(
            collective_id=0,
            vmem_limit_bytes=100 * 1024 * 1024,
        ),
    )(partial)


# device time: 1464064 ns/iter; 1.8909x vs baseline; 1.8909x over previous
import jax
import jax.numpy as jnp
from jax import lax
from jax.experimental import pallas as pl
from jax.experimental.pallas import tpu as pltpu

N_DEV = 8

RING = [0, 1, 2, 3, 7, 6, 5, 4]
POS_BY_ID = [RING.index(i) for i in range(N_DEV)]
RIGHT_BY_ID = [RING[(RING.index(i) + 1) % N_DEV] for i in range(N_DEV)]
LEFT_BY_ID = [RING[(RING.index(i) - 1) % N_DEV] for i in range(N_DEV)]


def _lut(idx, table):
    out = jnp.int32(table[0])
    for k in range(1, len(table)):
        out = jnp.where(idx == k, jnp.int32(table[k]), out)
    return out


def _gelu(y):
    c = 0.7978845608028654
    return 0.5 * y * (1.0 + jnp.tanh(c * (y + 0.044715 * y * y * y)))


def kernel(x, w_mat):
    partial = jnp.dot(x, w_mat, preferred_element_type=jnp.float32)
    m, n = partial.shape
    ch = m // N_DEV
    hf = ch // 2

    def body(p_ref, out_ref, buf_a, buf_b, local_a, local_b,
             send_sems_a, recv_sems_a, send_sems_b, recv_sems_b,
             load_sem_a, load_sem_b, store_sem_a, store_sem_b,
             ack_a, ack_b):
        me = lax.axis_index("i")
        r = _lut(me, POS_BY_ID)
        right = _lut(me, RIGHT_BY_ID)
        left = _lut(me, LEFT_BY_ID)

        barrier_sem = pltpu.get_barrier_semaphore()
        for nbr in (left, right):
            pl.semaphore_signal(barrier_sem, inc=1, device_id=(nbr,),
                                device_id_type=pl.DeviceIdType.MESH)
        pl.semaphore_wait(barrier_sem, 2)

        def top(c):
            return pl.ds(c * ch, hf)

        def bot(c):
            return pl.ds(c * ch + hf, hf)

        def rdma(buf, cur, nxt, sends, recvs, k, dst):
            return pltpu.make_async_remote_copy(
                src_ref=buf.at[cur], dst_ref=buf.at[nxt],
                send_sem=sends.at[k], recv_sem=recvs.at[k],
                device_id=(dst,), device_id_type=pl.DeviceIdType.MESH)

        def signal(sem, dst):
            pl.semaphore_signal(sem, inc=1, device_id=(dst,),
                                device_id_type=pl.DeviceIdType.MESH)

        cp_a = pltpu.make_async_copy(p_ref.at[top(r)], buf_a.at[0], load_sem_a)
        cp_b = pltpu.make_async_copy(p_ref.at[bot(r)], buf_b.at[0], load_sem_b)
        cp_a.start()
        cp_b.start()
        cp_a.wait()
        cp_b.wait()

        for s in range(N_DEV - 1):
            cur, nxt = s % 2, (s + 1) % 2
            rd_a = rdma(buf_a, cur, nxt, send_sems_a, recv_sems_a, s, right)
            rd_b = rdma(buf_b, cur, nxt, send_sems_b, recv_sems_b, s, left)
            if s > 0:
                pl.semaphore_wait(ack_a, 1)
                pl.semaphore_wait(ack_b, 1)
            rd_a.start()
            rd_b.start()
            ca = jnp.mod(r - s - 1, N_DEV)
            cb = jnp.mod(r + s + 1, N_DEV)
            cp_a = pltpu.make_async_copy(p_ref.at[top(ca)], local_a, load_sem_a)
            cp_b = pltpu.make_async_copy(p_ref.at[bot(cb)], local_b, load_sem_b)
            cp_a.start()
            cp_b.start()
            rd_a.wait()
            rd_b.wait()
            signal(ack_a, left)
            signal(ack_b, right)
            cp_a.wait()
            cp_b.wait()
            buf_a[nxt] = buf_a[nxt] + local_a[...]
            buf_b[nxt] = buf_b[nxt] + local_b[...]

        buf_a[1] = _gelu(buf_a[1])
        buf_b[1] = _gelu(buf_b[1])
        st_a = pltpu.make_async_copy(
            buf_a.at[1], out_ref.at[top(jnp.mod(r + 1, N_DEV))], store_sem_a)
        st_b = pltpu.make_async_copy(
            buf_b.at[1], out_ref.at[bot(jnp.mod(r - 1, N_DEV))], store_sem_b)
        st_a.start()
        st_b.start()

        for t in range(N_DEV - 1):
            cur, nxt = (1 + t) % 2, t % 2
            rd_a = rdma(buf_a, cur, nxt, send_sems_a, recv_sems_a,
                        N_DEV - 1 + t, right)
            rd_b = rdma(buf_b, cur, nxt, send_sems_b, recv_sems_b,
                        N_DEV - 1 + t, left)
            pl.semaphore_wait(ack_a, 1)
            pl.semaphore_wait(ack_b, 1)
            rd_a.start()
            rd_b.start()
            rd_a.wait()
            rd_b.wait()
            st_a.wait()
            st_b.wait()
            if t < N_DEV - 2:
                signal(ack_a, left)
                signal(ack_b, right)
            st_a = pltpu.make_async_copy(
                buf_a.at[nxt], out_ref.at[top(jnp.mod(r - t, N_DEV))],
                store_sem_a)
            st_b = pltpu.make_async_copy(
                buf_b.at[nxt], out_ref.at[bot(jnp.mod(r + t, N_DEV))],
                store_sem_b)
            st_a.start()
            st_b.start()
        st_a.wait()
        st_b.wait()

    n_sems = 2 * (N_DEV - 1)
    return pl.pallas_call(
        body,
        out_shape=jax.ShapeDtypeStruct((m, n), jnp.float32),
        in_specs=[pl.BlockSpec(memory_space=pltpu.MemorySpace.HBM)],
        out_specs=pl.BlockSpec(memory_space=pltpu.MemorySpace.HBM),
        scratch_shapes=[
            pltpu.VMEM((2, hf, n), jnp.float32),
            pltpu.VMEM((2, hf, n), jnp.float32),
            pltpu.VMEM((hf, n), jnp.float32),
            pltpu.VMEM((hf, n), jnp.float32),
            pltpu.SemaphoreType.DMA((n_sems,)),
            pltpu.SemaphoreType.DMA((n_sems,)),
            pltpu.SemaphoreType.DMA((n_sems,)),
            pltpu.SemaphoreType.DMA((n_sems,)),
            pltpu.SemaphoreType.DMA,
            pltpu.SemaphoreType.DMA,
            pltpu.SemaphoreType.DMA,
            pltpu.SemaphoreType.DMA,
            pltpu.SemaphoreType.REGULAR,
            pltpu.SemaphoreType.REGULAR,
        ],
        compiler_params=pltpu.CompilerParams(
            collective_id=0,
            vmem_limit_bytes=100 * 1024 * 1024,
        ),
    )(partial)


# device time: 1411709 ns/iter; 1.9611x vs baseline; 1.0371x over previous
import jax
import jax.numpy as jnp
from jax import lax
from jax.experimental import pallas as pl
from jax.experimental.pallas import tpu as pltpu

N_DEV = 8

RING = [0, 1, 2, 3, 7, 6, 5, 4]
POS_BY_ID = [RING.index(i) for i in range(N_DEV)]
RIGHT_BY_ID = [RING[(RING.index(i) + 1) % N_DEV] for i in range(N_DEV)]
LEFT_BY_ID = [RING[(RING.index(i) - 1) % N_DEV] for i in range(N_DEV)]

WT = 2048
N_WT = 8192 // WT


def _lut(idx, table):
    out = jnp.int32(table[0])
    for k in range(1, len(table)):
        out = jnp.where(idx == k, jnp.int32(table[k]), out)
    return out


def _gelu(y):
    c = 0.7978845608028654
    return 0.5 * y * (1.0 + jnp.tanh(c * (y + 0.044715 * y * y * y)))


def kernel(x, w_mat):
    m = x.shape[0]
    n = w_mat.shape[1]
    ch = m // N_DEV
    hf = ch // 2

    def body(x_ref, w_ref, out_ref, buf_a, buf_b, local_a, local_b,
             xa_buf, xb_buf, w_stage,
             send_sems_a, recv_sems_a, send_sems_b, recv_sems_b,
             xa_sem, xb_sem, w_sems, store_sem_a, store_sem_b,
             ack_a, ack_b):
        me = lax.axis_index("i")
        r = _lut(me, POS_BY_ID)
        right = _lut(me, RIGHT_BY_ID)
        left = _lut(me, LEFT_BY_ID)

        barrier_sem = pltpu.get_barrier_semaphore()
        for nbr in (left, right):
            pl.semaphore_signal(barrier_sem, inc=1, device_id=(nbr,),
                                device_id_type=pl.DeviceIdType.MESH)
        pl.semaphore_wait(barrier_sem, 2)

        def top(c):
            return pl.ds(c * ch, hf)

        def bot(c):
            return pl.ds(c * ch + hf, hf)

        def rdma(buf, cur, nxt, sends, recvs, k, dst):
            return pltpu.make_async_remote_copy(
                src_ref=buf.at[cur], dst_ref=buf.at[nxt],
                send_sem=sends.at[k], recv_sem=recvs.at[k],
                device_id=(dst,), device_id_type=pl.DeviceIdType.MESH)

        def signal(sem, dst):
            pl.semaphore_signal(sem, inc=1, device_id=(dst,),
                                device_id_type=pl.DeviceIdType.MESH)

        def start_x_loads(ca, cb):
            cp_a = pltpu.make_async_copy(x_ref.at[top(ca)], xa_buf, xa_sem)
            cp_b = pltpu.make_async_copy(x_ref.at[bot(cb)], xb_buf, xb_sem)
            cp_a.start()
            cp_b.start()
            return cp_a, cp_b

        def w_tile_copy(j):
            return pltpu.make_async_copy(
                w_ref.at[:, pl.ds(j * WT, WT)], w_stage.at[j % 2],
                w_sems.at[j % 2])

        def gemm_both(cp_a, cp_b, sink_a, sink_b):
            cp_a.wait()
            cp_b.wait()
            wc = w_tile_copy(0)
            for j in range(N_WT):
                wc.wait()
                if j + 1 < N_WT:
                    wc = w_tile_copy(j + 1)
                    wc.start()
                wt = w_stage[j % 2]
                sink_a(j, jnp.dot(xa_buf[...], wt,
                                  preferred_element_type=jnp.float32))
                sink_b(j, jnp.dot(xb_buf[...], wt,
                                  preferred_element_type=jnp.float32))

        def col(j):
            return pl.ds(j * WT, WT)

        cp_a, cp_b = start_x_loads(r, r)
        w_tile_copy(0).start()

        def _sink_buf(buf):
            def sink(j, val):
                buf[0, :, col(j)] = val
            return sink

        gemm_both(cp_a, cp_b, _sink_buf(buf_a), _sink_buf(buf_b))

        for s in range(N_DEV - 1):
            cur, nxt = s % 2, (s + 1) % 2
            rd_a = rdma(buf_a, cur, nxt, send_sems_a, recv_sems_a, s, right)
            rd_b = rdma(buf_b, cur, nxt, send_sems_b, recv_sems_b, s, left)
            if s > 0:
                pl.semaphore_wait(ack_a, 1)
                pl.semaphore_wait(ack_b, 1)
            rd_a.start()
            rd_b.start()
            ca = jnp.mod(r - s - 1, N_DEV)
            cb = jnp.mod(r + s + 1, N_DEV)
            cp_a, cp_b = start_x_loads(ca, cb)
            w_tile_copy(0).start()

            def _sink_local(loc):
                def sink(j, val):
                    loc[:, col(j)] = val
                return sink

            gemm_both(cp_a, cp_b, _sink_local(local_a), _sink_local(local_b))
            rd_a.wait()
            rd_b.wait()
            signal(ack_a, left)
            signal(ack_b, right)
            buf_a[nxt] = buf_a[nxt] + local_a[...]
            buf_b[nxt] = buf_b[nxt] + local_b[...]

        buf_a[1] = _gelu(buf_a[1])
        buf_b[1] = _gelu(buf_b[1])
        st_a = pltpu.make_async_copy(
            buf_a.at[1], out_ref.at[top(jnp.mod(r + 1, N_DEV))], store_sem_a)
        st_b = pltpu.make_async_copy(
            buf_b.at[1], out_ref.at[bot(jnp.mod(r - 1, N_DEV))], store_sem_b)
        st_a.start()
        st_b.start()

        for t in range(N_DEV - 1):
            cur, nxt = (1 + t) % 2, t % 2
            rd_a = rdma(buf_a, cur, nxt, send_sems_a, recv_sems_a,
                        N_DEV - 1 + t, right)
            rd_b = rdma(buf_b, cur, nxt, send_sems_b, recv_sems_b,
                        N_DEV - 1 + t, left)
            pl.semaphore_wait(ack_a, 1)
            pl.semaphore_wait(ack_b, 1)
            rd_a.start()
            rd_b.start()
            rd_a.wait()
            rd_b.wait()
            st_a.wait()
            st_b.wait()
            if t < N_DEV - 2:
                signal(ack_a, left)
                signal(ack_b, right)
            st_a = pltpu.make_async_copy(
                buf_a.at[nxt], out_ref.at[top(jnp.mod(r - t, N_DEV))],
                store_sem_a)
            st_b = pltpu.make_async_copy(
                buf_b.at[nxt], out_ref.at[bot(jnp.mod(r + t, N_DEV))],
                store_sem_b)
            st_a.start()
            st_b.start()
        st_a.wait()
        st_b.wait()

    n_sems = 2 * (N_DEV - 1)
    return pl.pallas_call(
        body,
        out_shape=jax.ShapeDtypeStruct((m, n), jnp.float32),
        in_specs=[
            pl.BlockSpec(memory_space=pltpu.MemorySpace.HBM),
            pl.BlockSpec(memory_space=pltpu.MemorySpace.HBM),
        ],
        out_specs=pl.BlockSpec(memory_space=pltpu.MemorySpace.HBM),
        scratch_shapes=[
            pltpu.VMEM((2, hf, n), jnp.float32),
            pltpu.VMEM((2, hf, n), jnp.float32),
            pltpu.VMEM((hf, n), jnp.float32),
            pltpu.VMEM((hf, n), jnp.float32),
            pltpu.VMEM((hf, x.shape[1]), jnp.float32),
            pltpu.VMEM((hf, x.shape[1]), jnp.float32),
            pltpu.VMEM((2, x.shape[1], WT), jnp.float32),
            pltpu.SemaphoreType.DMA((n_sems,)),
            pltpu.SemaphoreType.DMA((n_sems,)),
            pltpu.SemaphoreType.DMA((n_sems,)),
            pltpu.SemaphoreType.DMA((n_sems,)),
            pltpu.SemaphoreType.DMA,
            pltpu.SemaphoreType.DMA,
            pltpu.SemaphoreType.DMA((2,)),
            pltpu.SemaphoreType.DMA,
            pltpu.SemaphoreType.DMA,
            pltpu.SemaphoreType.REGULAR,
            pltpu.SemaphoreType.REGULAR,
        ],
        compiler_params=pltpu.CompilerParams(
            collective_id=0,
            vmem_limit_bytes=110 * 1024 * 1024,
        ),
    )(x, w_mat)
